# baseline (speedup 1.0000x reference)
import functools

import jax
import jax.numpy as jnp
from jax import lax
from jax.experimental import pallas as pl
from jax.experimental.pallas import tpu as pltpu
from jax.experimental.pallas import tpu_sc as plsc

NUM_EMB = 1000000
D = 64
QW = NUM_EMB // 4
T = 4096 * 50
NW = 32
C = 128
NBLK = T // (NW * C)
IBLK = 4096 // C

QB = NUM_EMB * D
BPW = QB // NW
CH = 40000
NCH = BPW // CH
assert NCH * CH == BPW and CH % 64 == 0 and (CH // 4) % 8 == 0


def _cbody(q8_ref, out_ref, buf8, buf32, sem):
    nc = 2
    wid = lax.axis_index("s") * nc + lax.axis_index("c")
    base = wid * BPW

    @pl.loop(0, NCH)
    def chunk_body(k):
        off = base + k * CH
        pltpu.sync_copy(q8_ref.at[pl.ds(off, CH)], buf8)

        @pl.loop(0, CH // 64)
        def conv_body(a):
            w = plsc.bitcast(buf8[pl.ds(a * 64, 64)], jnp.int32)
            buf32[pl.ds(a * 16, 16)] = w

        woff = pl.multiple_of(off // 4, 8)
        pltpu.sync_copy(buf32, out_ref.at[pl.ds(woff, CH // 4)])


_conv_call = functools.partial(
    pl.kernel,
    out_type=jax.ShapeDtypeStruct((QB // 4,), jnp.int32),
    mesh=plsc.VectorSubcoreMesh(core_axis_name="c", subcore_axis_name="s"),
    compiler_params=pltpu.CompilerParams(
        needs_layout_passes=False, use_tc_tiling_on_sc=False),
    scratch_types=[
        pltpu.VMEM((CH,), jnp.int8),
        pltpu.VMEM((CH // 4,), jnp.int32),
        pltpu.SemaphoreType.DMA,
    ],
)(_cbody)


def _body(x_ref, q_ref, s0_ref, s1_ref, z_ref, out_ref,
          idx_v, shl_v, idxw, qw, zv, sv0, sv1, zf0, zf1, deq, semq, sems):
    nc = 2
    wid = lax.axis_index("s") * nc + lax.axis_index("c")

    @pl.loop(0, NBLK)
    def block_body(k):
        b = wid * NBLK + k
        j = b // IBLK
        i0 = (b % IBLK) * C
        pltpu.sync_copy(x_ref.at[pl.ds(j * 4096 + i0, C)], idx_v)

        @pl.loop(0, C // 16)
        def pre_body(g):
            iv = idx_v[pl.ds(g * 16, 16)]
            idxw[0, pl.ds(g * 16, 16)] = iv >> 2
            shl_v[pl.ds(g * 16, 16)] = (3 - (iv & 3)) * 8

        @pl.loop(1, D)
        def colidx_body(c):
            off = c * QW
            for g in range(C // 16):
                sl = pl.ds(g * 16, 16)
                idxw[c, sl] = idxw[0, sl] + off

        cpz = pltpu.async_copy(z_ref.at[idx_v], zv, sems)
        cps0 = pltpu.async_copy(s0_ref.at[idx_v], sv0, sems)
        cps1 = pltpu.async_copy(s1_ref.at[idx_v], sv1, sems)

        @pl.loop(0, D)
        def gather_body(c):
            pltpu.async_copy(q_ref.at[idxw.at[c]], qw.at[pl.ds(c * C, C)],
                             semq)

        cpz.wait()
        cps0.wait()
        cps1.wait()

        @pl.loop(0, C // 16)
        def zpre_body(g):
            zw = zv[pl.ds(g * 16, 16)]
            zf0[pl.ds(g * 16, 16)] = ((zw << 24) >> 24).astype(jnp.float32)
            zf1[pl.ds(g * 16, 16)] = ((zw << 16) >> 24).astype(jnp.float32)

        pltpu.make_async_copy(q_ref.at[pl.ds(0, D * C)], qw, semq).wait()

        @pl.loop(0, C // 16)
        def grp_body(g):
            sl = pl.ds(g * 16, 16)
            shl16 = shl_v[sl]
            s0_16 = sv0[sl]
            s1_16 = sv1[sl]
            z0_16 = zf0[sl]
            z1_16 = zf1[sl]

            @pl.loop(0, D // 2)
            def c_body0(c):
                w = qw[pl.ds(c * C + g * 16, 16)]
                v = ((w << shl16) >> 24).astype(jnp.float32)
                deq[c, sl] = (v - z0_16) * s0_16

            @pl.loop(D // 2, D)
            def c_body1(c):
                w = qw[pl.ds(c * C + g * 16, 16)]
                v = ((w << shl16) >> 24).astype(jnp.float32)
                deq[c, sl] = (v - z1_16) * s1_16

        pltpu.sync_copy(deq, out_ref.at[j, :, pl.ds(i0, C)])


_sc_call = functools.partial(
    pl.kernel,
    out_type=jax.ShapeDtypeStruct((50, D, 4096), jnp.float32),
    mesh=plsc.VectorSubcoreMesh(core_axis_name="c", subcore_axis_name="s"),
    compiler_params=pltpu.CompilerParams(
        needs_layout_passes=False, use_tc_tiling_on_sc=False),
    scratch_types=[
        pltpu.VMEM((C,), jnp.int32),
        pltpu.VMEM((C,), jnp.int32),
        pltpu.VMEM((D, C), jnp.int32),
        pltpu.VMEM((D * C,), jnp.int32),
        pltpu.VMEM((C,), jnp.int32),
        pltpu.VMEM((C,), jnp.float32),
        pltpu.VMEM((C,), jnp.float32),
        pltpu.VMEM((C,), jnp.float32),
        pltpu.VMEM((C,), jnp.float32),
        pltpu.VMEM((D, C), jnp.float32),
        pltpu.SemaphoreType.DMA,
        pltpu.SemaphoreType.DMA,
    ],
)(_body)


@jax.jit
def kernel(x, qvals, scales, zeros):
    xf = x.T.reshape(-1)
    qtab = _conv_call(qvals.T.reshape(-1))
    stab0 = scales.T[0]
    stab1 = scales.T[1]
    z32 = lax.bitcast_convert_type(zeros, jnp.int16).astype(jnp.int32)
    out3 = _sc_call(xf, qtab, stab0, stab1, z32)
    return out3.transpose(2, 0, 1)


# speedup vs baseline: 9.4507x; 9.4507x over previous
import functools

import jax
import jax.numpy as jnp
from jax import lax
from jax.experimental import pallas as pl
from jax.experimental.pallas import tpu as pltpu
from jax.experimental.pallas import tpu_sc as plsc

NUM_EMB = 1000000
D = 64
DW = D // 4
G = 2
T = 4096 * 50
NW = 32
N_PER = T // NW
C = 128
NCHUNK = N_PER // C


def _body(x_ref, q_ref, s0_ref, s1_ref, z_ref, out_ref,
          idx_v, qv, sv0, sv1, zv, zf, ov, sem):
    nc = 2
    wid = lax.axis_index("s") * nc + lax.axis_index("c")
    base = wid * N_PER

    lane = lax.iota(jnp.int32, 16)
    m8 = lane < 8
    halfsel = (lane >= 8).astype(jnp.int32)
    e2 = lane * 2
    cols = [lane * 4 + k for k in range(4)]

    def chunk_body(ci, carry):
        cbase = base + ci * C
        pltpu.sync_copy(x_ref.at[pl.ds(cbase, C)], idx_v)
        cp_q = pltpu.async_copy(q_ref.at[idx_v], qv, sem)
        cp_s0 = pltpu.async_copy(s0_ref.at[idx_v], sv0, sem)
        cp_s1 = pltpu.async_copy(s1_ref.at[idx_v], sv1, sem)
        cp_z = pltpu.async_copy(z_ref.at[idx_v], zv, sem)
        cp_q.wait()
        cp_s0.wait()
        cp_s1.wait()
        cp_z.wait()

        def pre_body(j, pcarry):
            zw = zv[pl.ds(j * 16, 16)]
            z0 = ((zw << 24) >> 24).astype(jnp.float32)
            z1 = ((zw << 16) >> 24).astype(jnp.float32)
            pbase = jnp.broadcast_to(j * 32, (16,))
            plsc.store_scatter(zf, [pbase + e2], z0)
            plsc.store_scatter(zf, [pbase + e2 + 1], z1)
            return pcarry

        lax.fori_loop(0, C // 16, pre_body, 0)

        def row_body(i, rcarry):
            w = plsc.bitcast(qv[i, :], jnp.int32)
            b0 = (w << 24) >> 24
            b1 = (w << 16) >> 24
            b2 = (w << 8) >> 24
            b3 = w >> 24
            rowv = jnp.broadcast_to(i, (16,))
            idx_sz = jnp.broadcast_to(i * 2, (16,)) + halfsel
            svec = jnp.where(m8, plsc.load_gather(sv0, [rowv]),
                             plsc.load_gather(sv1, [rowv]))
            zvec = plsc.load_gather(zf, [idx_sz])
            obase = jnp.broadcast_to(i * 64, (16,))
            for k, bk in enumerate((b0, b1, b2, b3)):
                fk = bk.astype(jnp.float32)
                plsc.store_scatter(ov, [obase + cols[k]], (fk - zvec) * svec)
            return rcarry

        lax.fori_loop(0, C, row_body, 0)
        pltpu.sync_copy(ov, out_ref.at[pl.ds(cbase * D, C * D)])
        return carry

    lax.fori_loop(0, NCHUNK, chunk_body, 0)


_sc_call = functools.partial(
    pl.kernel,
    out_type=jax.ShapeDtypeStruct((T * D,), jnp.float32),
    mesh=plsc.VectorSubcoreMesh(core_axis_name="c", subcore_axis_name="s"),
    compiler_params=pltpu.CompilerParams(
        needs_layout_passes=False, use_tc_tiling_on_sc=False),
    scratch_types=[
        pltpu.VMEM((C,), jnp.int32),
        pltpu.VMEM((C, D), jnp.int8),
        pltpu.VMEM((C,), jnp.float32),
        pltpu.VMEM((C,), jnp.float32),
        pltpu.VMEM((C,), jnp.int32),
        pltpu.VMEM((C * G,), jnp.float32),
        pltpu.VMEM((C * D,), jnp.float32),
        pltpu.SemaphoreType.DMA,
    ],
)(_body)


@jax.jit
def kernel(x, qvals, scales, zeros):
    xf = x.reshape(-1)
    z32 = lax.bitcast_convert_type(zeros, jnp.int16).astype(jnp.int32)
    out = _sc_call(xf, qvals, scales.T[0], scales.T[1], z32)
    return out.reshape(*x.shape, D)
